# baseline (speedup 1.0000x reference)
import functools

import jax
import jax.numpy as jnp
from jax import lax
from jax.experimental import pallas as pl
from jax.experimental.pallas import tpu as pltpu
from jax.experimental.pallas import tpu_sc as plsc

B = 16384
L = 50
VOCAB = 1000
VP = 1024
EMB = 125
D = 128

NC = 2
NS = 16
NW = NC * NS
ROWS_PER_W = B // NW
GR = 2
NGRP = ROWS_PER_W // GR
NSLOT = 4
HB = NSLOT * GR
NPI = NGRP // (2 * NSLOT)
NJ = D // 16


def _body(tok_hbm, loc_hbm, table_hbm, out_hbm,
          table_sh, tok_v, loc_v, rows_v,
          out_a, out_b, tsem, lsem, osem, gsem0, gsem1, gsem2, gsem3):
    cid = lax.axis_index("c")
    sid = lax.axis_index("s")
    wid = sid * NC + cid
    gsems = (gsem0, gsem1, gsem2, gsem3)

    @pl.when(sid == 0)
    def _stage():
        pltpu.sync_copy(table_hbm, table_sh)

    plsc.subcore_barrier()

    pltpu.make_async_copy(
        tok_hbm.at[pl.ds(wid * NGRP, NGRP)], tok_v, tsem).start()
    pltpu.make_async_copy(
        loc_hbm.at[pl.ds(wid * (ROWS_PER_W // 8), ROWS_PER_W // 8)],
        loc_v, lsem).start()
    pltpu.make_async_copy(
        tok_hbm.at[pl.ds(wid * NGRP, NGRP)], tok_v, tsem).wait()
    pltpu.make_async_copy(
        loc_hbm.at[pl.ds(wid * (ROWS_PER_W // 8), ROWS_PER_W // 8)],
        loc_v, lsem).wait()

    def gath(grp, slot):
        return pltpu.make_async_copy(
            table_sh.at[tok_v.at[grp]], rows_v.at[slot], gsems[slot])

    def out_copy(row0, out_v):
        return pltpu.make_async_copy(
            out_v, out_hbm.at[pl.ds(wid * ROWS_PER_W + row0, HB)], osem)

    for k in range(NSLOT):
        gath(k, k).start()

    def sum_group(grp, slot, orow, out_v):
        for r2 in range(GR):
            def tsum(t, acc):
                return tuple(
                    a + rows_v[slot, r2 * L + t, pl.ds(j * 16, 16)]
                    for j, a in enumerate(acc)
                )

            acc = list(lax.fori_loop(
                0, L, tsum,
                tuple(jnp.zeros((16,), jnp.float32) for _ in range(NJ)),
                unroll=10,
            ))
            lr = grp * GR + r2
            acc[NJ - 1] = acc[NJ - 1] + loc_v[
                lax.shift_right_logical(lr, 3),
                pl.ds(lax.bitwise_and(lr, 7) * 16, 16)]
            for j in range(NJ):
                out_v[orow + r2, pl.ds(j * 16, 16)] = acc[j]

    def half(pi, h, out_v):
        gb = pi * (2 * NSLOT) + h * NSLOT

        @pl.when(pi > 0)
        def _drain_prev():
            out_copy(0, out_v).wait()

        for k in range(NSLOT):
            grp = gb + k
            gath(grp, k).wait()
            sum_group(grp, k, k * GR, out_v)

            @pl.when(grp + NSLOT < NGRP)
            def _rearm():
                gath(grp + NSLOT, k).start()

        out_copy(gb * GR, out_v).start()

    def pair_body(pi, _):
        half(pi, 0, out_a)
        half(pi, 1, out_b)
        return _

    lax.fori_loop(0, NPI, pair_body, None)
    out_copy(0, out_a).wait()
    out_copy(0, out_b).wait()


@jax.jit
def _sc_pool(tok2, loc_p, table_p):
    return pl.kernel(
        _body,
        out_type=jax.ShapeDtypeStruct((B, D), jnp.float32),
        mesh=plsc.VectorSubcoreMesh(core_axis_name="c", subcore_axis_name="s"),
        scratch_types=[
            pltpu.VMEM_SHARED((VP, D), jnp.float32),
            pltpu.VMEM((NGRP, GR * L), jnp.int32),
            pltpu.VMEM((ROWS_PER_W // 8, 128), jnp.float32),
            pltpu.VMEM((NSLOT, GR * L, D), jnp.float32),
            pltpu.VMEM((HB, D), jnp.float32),
            pltpu.VMEM((HB, D), jnp.float32),
            pltpu.SemaphoreType.DMA,
            pltpu.SemaphoreType.DMA,
            pltpu.SemaphoreType.DMA,
            pltpu.SemaphoreType.DMA,
            pltpu.SemaphoreType.DMA,
            pltpu.SemaphoreType.DMA,
            pltpu.SemaphoreType.DMA,
        ],
    )(tok2, loc_p, table_p)


def kernel(tokens, locsize, table):
    tok2 = tokens.astype(jnp.int32).reshape(B // GR, GR * L)
    table_p = jnp.zeros((VP, D), jnp.float32).at[:VOCAB, :EMB].set(table)
    loc_p = jnp.zeros((B, 16), jnp.float32).at[:, 13:].set(
        locsize).reshape(B // 8, 128)
    out = _sc_pool(tok2, loc_p, table_p)
    return out[:, None, :]
